# baseline (speedup 1.0000x reference)
import jax
import jax.numpy as jnp
from jax.experimental import pallas as pl
from jax.experimental.pallas import tpu as pltpu

_PAD = 0
_VMEM = 64 * 1024 * 1024


def _gru_body(x_ref, wih_ref, whh_ref, bgi_ref, bhn_ref, out_ref,
              gif_ref, gib_ref):
    T, TB, I = x_ref.shape
    H = whh_ref.shape[1]
    x2 = x_ref[...].reshape(T * TB, I)
    gif_ref[...] = (jnp.dot(x2, wih_ref[0], preferred_element_type=jnp.float32)
                    + bgi_ref[0]).reshape(T, TB, 3 * H)
    gib_ref[...] = (jnp.dot(x2, wih_ref[1], preferred_element_type=jnp.float32)
                    + bgi_ref[1]).reshape(T, TB, 3 * H)

    whh_f = whh_ref[0]
    whh_b = whh_ref[1]
    bhn_f = bhn_ref[0]
    bhn_b = bhn_ref[1]

    def gate(gi_t, gh, bhn, h):
        rz = jax.nn.sigmoid(gi_t[:, :2 * H] + gh[:, :2 * H])
        r = rz[:, :H]
        z = rz[:, H:]
        n = jnp.tanh(gi_t[:, 2 * H:] + r * (gh[:, 2 * H:] + bhn))
        return n + z * (h - n)

    hf = jnp.zeros((TB, H), jnp.float32)
    hb = jnp.zeros((TB, H), jnp.float32)
    hf16 = hf.astype(jnp.bfloat16)
    hb16 = hb.astype(jnp.bfloat16)
    for s in range(T):
        tb = T - 1 - s
        ghf = jnp.dot(hf16, whh_f, preferred_element_type=jnp.float32)
        ghb = jnp.dot(hb16, whh_b, preferred_element_type=jnp.float32)
        hf = gate(gif_ref[s], ghf, bhn_f, hf)
        hb = gate(gib_ref[tb], ghb, bhn_b, hb)
        hf16 = hf.astype(jnp.bfloat16)
        hb16 = hb.astype(jnp.bfloat16)
        out_ref[s, :, :H] = hf16
        out_ref[tb, :, H:] = hb16


def _gru_layer(x, wih, whh, bgi, bhn):
    T, B, I = x.shape
    H = whh.shape[1]
    TB = B // 2
    return pl.pallas_call(
        _gru_body,
        out_shape=jax.ShapeDtypeStruct((T, B, 2 * H), jnp.bfloat16),
        grid=(2,),
        in_specs=[
            pl.BlockSpec((T, TB, I), lambda c: (0, c, 0)),
            pl.BlockSpec((2, I, 3 * H), lambda c: (0, 0, 0)),
            pl.BlockSpec((2, H, 3 * H), lambda c: (0, 0, 0)),
            pl.BlockSpec((2, 1, 3 * H), lambda c: (0, 0, 0)),
            pl.BlockSpec((2, 1, H), lambda c: (0, 0, 0)),
        ],
        out_specs=pl.BlockSpec((T, TB, 2 * H), lambda c: (0, c, 0)),
        scratch_shapes=[pltpu.VMEM((T, TB, 3 * H), jnp.float32),
                        pltpu.VMEM((T, TB, 3 * H), jnp.float32)],
        compiler_params=pltpu.CompilerParams(
            dimension_semantics=("parallel",),
            vmem_limit_bytes=_VMEM),
    )(x, wih, whh, bgi, bhn)


def _attn_body(inp_ref, mask_ref, ws1_ref, ws2_ref, fcw_ref, fcb_ref,
               pw_ref, pb_ref, pred_ref, attn_ref):
    TB, T, D2 = inp_ref.shape
    hops = ws2_ref.shape[1]
    inp = inp_ref[...]
    inp2 = inp.reshape(TB * T, D2)
    hbar = jnp.tanh(jnp.dot(inp2, ws1_ref[...],
                            preferred_element_type=jnp.float32))
    scores = jnp.dot(hbar.astype(jnp.bfloat16), ws2_ref[...],
                     preferred_element_type=jnp.float32)
    alphas = jnp.swapaxes(scores.reshape(TB, T, hops), 1, 2)
    pen = alphas - 10000.0 * mask_ref[...]
    m = jnp.max(pen, axis=-1, keepdims=True)
    e = jnp.exp(pen - m)
    a = e / jnp.sum(e, axis=-1, keepdims=True)
    attn_ref[...] = a
    agg = jnp.einsum("bht,btd->bhd", a.astype(jnp.bfloat16), inp,
                     preferred_element_type=jnp.float32)
    flat = agg.reshape(TB, hops * D2).astype(jnp.bfloat16)
    fc = jnp.tanh(jnp.dot(flat, fcw_ref[...],
                          preferred_element_type=jnp.float32) + fcb_ref[...])
    pred = jnp.dot(fc.astype(jnp.bfloat16), pw_ref[...],
                   preferred_element_type=jnp.float32) + pb_ref[...]
    pred_ref[...] = pred


def _attn_classifier(inp, mask, ws1, ws2, fcw, fcb, pw, pb):
    B, T, D2 = inp.shape
    A = ws1.shape[1]
    hops = ws2.shape[1]
    nfc = fcw.shape[1]
    ncls = pw.shape[1]
    TB = B // 2
    z2 = lambda b: (0, 0)
    return pl.pallas_call(
        _attn_body,
        out_shape=(jax.ShapeDtypeStruct((B, ncls), jnp.float32),
                   jax.ShapeDtypeStruct((B, hops, T), jnp.float32)),
        grid=(2,),
        in_specs=[
            pl.BlockSpec((TB, T, D2), lambda b: (b, 0, 0)),
            pl.BlockSpec((TB, 1, T), lambda b: (b, 0, 0)),
            pl.BlockSpec((D2, A), z2),
            pl.BlockSpec((A, hops), z2),
            pl.BlockSpec((hops * D2, nfc), z2),
            pl.BlockSpec((1, nfc), z2),
            pl.BlockSpec((nfc, ncls), z2),
            pl.BlockSpec((1, ncls), z2),
        ],
        out_specs=(pl.BlockSpec((TB, ncls), lambda b: (b, 0)),
                   pl.BlockSpec((TB, hops, T), lambda b: (b, 0, 0))),
        compiler_params=pltpu.CompilerParams(
            dimension_semantics=("parallel",),
            vmem_limit_bytes=_VMEM),
    )(inp, mask, ws1, ws2, fcw, fcb, pw, pb)


def _fold_bias(bih, bhh):
    H3 = bih.shape[-1]
    H = H3 // 3
    bgi = bih.at[:, :, :2 * H].add(bhh[:, :, :2 * H])
    bhn = bhh[:, :, 2 * H:]
    return bgi, bhn


def kernel(tokens, emb, gru0_wih, gru0_whh, gru0_bih, gru0_bhh,
           gru1_wih, gru1_whh, gru1_bih, gru1_bhh,
           ws1, ws2, fcw, fcb, pw, pb):
    T, B = tokens.shape
    x = emb[tokens].astype(jnp.bfloat16)

    bgi0, bhn0 = _fold_bias(gru0_bih, gru0_bhh)
    bgi1, bhn1 = _fold_bias(gru1_bih, gru1_bhh)
    out0 = _gru_layer(x, gru0_wih.astype(jnp.bfloat16),
                      gru0_whh.astype(jnp.bfloat16), bgi0, bhn0)
    out1 = _gru_layer(out0, gru1_wih.astype(jnp.bfloat16),
                      gru1_whh.astype(jnp.bfloat16), bgi1, bhn1)

    inp = jnp.transpose(out1, (1, 0, 2))
    mask = (tokens.T == _PAD).astype(jnp.float32)[:, None, :]
    pred, attn = _attn_classifier(
        inp, mask, ws1.astype(jnp.bfloat16), ws2.astype(jnp.bfloat16),
        fcw.astype(jnp.bfloat16), fcb, pw.astype(jnp.bfloat16), pb)
    return pred, attn
